# baseline (speedup 1.0000x reference)
import jax
import jax.numpy as jnp
from jax.experimental import pallas as pl
from jax.experimental.pallas import tpu as pltpu

_BM = 200


def _gcn_block_kernel(a_ref, bias_ref, seq_ref, w_ref, adj_ref, out_ref,
                      sfts_ref):
    @pl.when(pl.program_id(0) == 0)
    def _():
        sfts = jax.lax.dot_general(
            seq_ref[...], w_ref[...],
            (((1,), (1,)), ((), ())),
            preferred_element_type=jnp.float32)
        sfts_ref[...] = sfts.astype(jnp.bfloat16)

    acc = jax.lax.dot_general(
        adj_ref[...].astype(jnp.bfloat16), sfts_ref[...],
        (((1,), (0,)), ((), ())),
        preferred_element_type=jnp.float32)
    acc = acc + bias_ref[...]
    a = a_ref[0, 0]
    out_ref[...] = jnp.where(acc >= 0, acc, a * acc)


def kernel(seq, adj, W, bias, prelu_a):
    n, in_ft = seq.shape
    out_ft = W.shape[0]
    a2 = jnp.reshape(prelu_a, (1, 1))
    bias2 = jnp.reshape(bias, (1, out_ft))
    grid = (n // _BM,)
    return pl.pallas_call(
        _gcn_block_kernel,
        grid=grid,
        in_specs=[
            pl.BlockSpec(memory_space=pltpu.SMEM),
            pl.BlockSpec((1, out_ft), lambda i: (0, 0)),
            pl.BlockSpec((n, in_ft), lambda i: (0, 0)),
            pl.BlockSpec((out_ft, in_ft), lambda i: (0, 0)),
            pl.BlockSpec((_BM, n), lambda i: (i, 0)),
        ],
        out_specs=pl.BlockSpec((_BM, out_ft), lambda i: (i, 0)),
        out_shape=jax.ShapeDtypeStruct((n, out_ft), jnp.float32),
        scratch_shapes=[pltpu.VMEM((n, out_ft), jnp.bfloat16)],
        compiler_params=pltpu.CompilerParams(
            dimension_semantics=("arbitrary",),
        ),
    )(a2, bias2, seq, W, adj)


# speedup vs baseline: 1.0108x; 1.0108x over previous
import jax
import jax.numpy as jnp
from jax.experimental import pallas as pl
from jax.experimental.pallas import tpu as pltpu

_BM = 400


def _gcn_block_kernel(a_ref, bias_ref, seq_ref, w_ref, adj_ref, out_ref,
                      sfts_ref):
    @pl.when(pl.program_id(0) == 0)
    def _():
        sfts = jax.lax.dot_general(
            seq_ref[...], w_ref[...],
            (((1,), (1,)), ((), ())),
            preferred_element_type=jnp.float32)
        sfts_ref[...] = sfts.astype(jnp.bfloat16)

    acc = jax.lax.dot_general(
        adj_ref[...].astype(jnp.bfloat16), sfts_ref[...],
        (((1,), (0,)), ((), ())),
        preferred_element_type=jnp.float32)
    acc = acc + bias_ref[...]
    a = a_ref[0, 0]
    out_ref[...] = jnp.where(acc >= 0, acc, a * acc)


def kernel(seq, adj, W, bias, prelu_a):
    n, in_ft = seq.shape
    out_ft = W.shape[0]
    a2 = jnp.reshape(prelu_a, (1, 1))
    bias2 = jnp.reshape(bias, (1, out_ft))
    grid = (n // _BM,)
    return pl.pallas_call(
        _gcn_block_kernel,
        grid=grid,
        in_specs=[
            pl.BlockSpec(memory_space=pltpu.SMEM),
            pl.BlockSpec((1, out_ft), lambda i: (0, 0)),
            pl.BlockSpec((n, in_ft), lambda i: (0, 0)),
            pl.BlockSpec((out_ft, in_ft), lambda i: (0, 0)),
            pl.BlockSpec((_BM, n), lambda i: (i, 0)),
        ],
        out_specs=pl.BlockSpec((_BM, out_ft), lambda i: (i, 0)),
        out_shape=jax.ShapeDtypeStruct((n, out_ft), jnp.float32),
        scratch_shapes=[pltpu.VMEM((n, out_ft), jnp.bfloat16)],
        compiler_params=pltpu.CompilerParams(
            dimension_semantics=("arbitrary",),
        ),
    )(a2, bias2, seq, W, adj)
